# baseline (speedup 1.0000x reference)
import jax
import jax.numpy as jnp
from jax import lax
from jax.experimental import pallas as pl
from jax.experimental.pallas import tpu as pltpu
from jax.experimental.pallas import tpu_sc as plsc

BATCH = 16384
EMBED = 64
PADW = 128
HID1 = 128
HID2 = 64
CHUNK = 128

_info = plsc.get_sparse_core_info()
_NC, _NS = _info.num_cores, _info.num_subcores
_NW = _NC * _NS
_BPW = BATCH // _NW
_NCHUNK = _BPW // CHUNK


def _gather_body(uidx_hbm, iidx_hbm, utab_hbm, itab_hbm, ue_hbm, ie_hbm,
                 idx_v, rows_v, sem):
    wid = lax.axis_index("s") * _NC + lax.axis_index("c")
    base = wid * _BPW
    row0 = wid * _NCHUNK

    def one_table(idx_hbm, tab_hbm, out_hbm):
        pltpu.sync_copy(idx_hbm.at[pl.ds(row0, _NCHUNK)], idx_v)
        copies = [
            pltpu.async_copy(tab_hbm.at[idx_v.at[j]],
                             rows_v.at[pl.ds(j * CHUNK, CHUNK)], sem)
            for j in range(_NCHUNK)
        ]
        for c in copies:
            c.wait()
        pltpu.sync_copy(rows_v, out_hbm.at[pl.ds(base, _BPW)])

    one_table(uidx_hbm, utab_hbm, ue_hbm)
    one_table(iidx_hbm, itab_hbm, ie_hbm)


_gather = pl.kernel(
    _gather_body,
    out_type=[
        jax.ShapeDtypeStruct((BATCH, PADW), jnp.float32),
        jax.ShapeDtypeStruct((BATCH, PADW), jnp.float32),
    ],
    mesh=plsc.VectorSubcoreMesh(core_axis_name="c", subcore_axis_name="s"),
    scratch_types=[
        pltpu.VMEM((_NCHUNK, CHUNK), jnp.int32),
        pltpu.VMEM((_BPW, PADW), jnp.float32),
        pltpu.SemaphoreType.DMA,
    ],
    compiler_params=pltpu.CompilerParams(use_tc_tiling_on_sc=True),
)


_TCOLS = 1024


def _tpose_body(tt, out):
    t = lax.dot_general(tt[...], jnp.eye(EMBED, dtype=jnp.float32),
                        (((0,), (0,)), ((), ())),
                        preferred_element_type=jnp.float32)
    out[...] = jnp.concatenate(
        [t, jnp.zeros((_TCOLS, PADW - EMBED), jnp.float32)], axis=1)


def _make_tpose(nrows):
    return pl.pallas_call(
        _tpose_body,
        grid=(pl.cdiv(nrows, _TCOLS),),
        in_specs=[pl.BlockSpec((EMBED, _TCOLS), lambda i: (0, i))],
        out_specs=pl.BlockSpec((_TCOLS, PADW), lambda i: (i, 0)),
        out_shape=jax.ShapeDtypeStruct((nrows, PADW), jnp.float32),
        compiler_params=pltpu.CompilerParams(
            dimension_semantics=("arbitrary",)),
    )


def _mlp_body(up, ip, w1a, w1b, b1, w2, b2, w3, b3, out):
    h = jnp.dot(up[:, :EMBED], w1a[...], preferred_element_type=jnp.float32)
    h = h + jnp.dot(ip[:, :EMBED], w1b[...], preferred_element_type=jnp.float32)
    h = jnp.maximum(h + b1[...], 0.0)
    h = jnp.maximum(
        jnp.dot(h, w2[...], preferred_element_type=jnp.float32) + b2[...], 0.0)
    o = jnp.dot(h, w3[...], preferred_element_type=jnp.float32)
    out[...] = o[:, 0] + b3[...][0, 0]


_BS = 2048

_mlp = pl.pallas_call(
    _mlp_body,
    grid=(BATCH // _BS,),
    in_specs=[
        pl.BlockSpec((_BS, PADW), lambda i: (i, 0)),
        pl.BlockSpec((_BS, PADW), lambda i: (i, 0)),
        pl.BlockSpec((EMBED, HID1), lambda i: (0, 0)),
        pl.BlockSpec((EMBED, HID1), lambda i: (0, 0)),
        pl.BlockSpec((1, HID1), lambda i: (0, 0)),
        pl.BlockSpec((HID1, HID2), lambda i: (0, 0)),
        pl.BlockSpec((1, HID2), lambda i: (0, 0)),
        pl.BlockSpec((HID2, 1), lambda i: (0, 0)),
        pl.BlockSpec((1, 1), lambda i: (0, 0)),
    ],
    out_specs=pl.BlockSpec((_BS,), lambda i: (i,)),
    out_shape=jax.ShapeDtypeStruct((BATCH,), jnp.float32),
    compiler_params=pltpu.CompilerParams(dimension_semantics=("arbitrary",)),
)


def kernel(users, items, user_table, item_table, W1, b1, W2, b2, W3, b3):
    uidx = users.astype(jnp.int32).reshape(BATCH // CHUNK, CHUNK)
    iidx = items.astype(jnp.int32).reshape(BATCH // CHUNK, CHUNK)
    tpose = _make_tpose(user_table.shape[0])
    tu = tpose(user_table.T)
    ti = tpose(item_table.T)
    up, ip = _gather(uidx, iidx, tu, ti)
    return _mlp(up, ip, W1[:EMBED], W1[EMBED:], b1.reshape(1, HID1),
                W2, b2.reshape(1, HID2), W3, b3.reshape(1, 1))


# speedup vs baseline: 2.8934x; 2.8934x over previous
import jax
import jax.numpy as jnp
from jax import lax
from jax.experimental import pallas as pl
from jax.experimental.pallas import tpu as pltpu
from jax.experimental.pallas import tpu_sc as plsc

BATCH = 16384
EMBED = 64
PADW = 128
HID1 = 128
HID2 = 64
CHUNK = 128

_info = plsc.get_sparse_core_info()
_NC, _NS = _info.num_cores, _info.num_subcores
_NW = _NC * _NS
_BPW = BATCH // _NW
_NCHUNK = _BPW // CHUNK


def _gather_body(uidx_hbm, iidx_hbm, utab_hbm, itab_hbm, ue_hbm, ie_hbm,
                 idx_v, rows_v, sem):
    wid = lax.axis_index("s") * _NC + lax.axis_index("c")
    base = wid * _BPW
    row0 = wid * _NCHUNK

    def one_table(idx_hbm, tab_hbm, out_hbm):
        pltpu.sync_copy(idx_hbm.at[pl.ds(row0, _NCHUNK)], idx_v)
        copies = [
            pltpu.async_copy(tab_hbm.at[idx_v.at[j]],
                             rows_v.at[pl.ds(j * CHUNK, CHUNK)], sem)
            for j in range(_NCHUNK)
        ]
        for c in copies:
            c.wait()
        pltpu.sync_copy(rows_v, out_hbm.at[pl.ds(base, _BPW)])

    one_table(uidx_hbm, utab_hbm, ue_hbm)
    one_table(iidx_hbm, itab_hbm, ie_hbm)


_gather = pl.kernel(
    _gather_body,
    out_type=[
        jax.ShapeDtypeStruct((BATCH, PADW), jnp.float32),
        jax.ShapeDtypeStruct((BATCH, PADW), jnp.float32),
    ],
    mesh=plsc.VectorSubcoreMesh(core_axis_name="c", subcore_axis_name="s"),
    scratch_types=[
        pltpu.VMEM((_NCHUNK, CHUNK), jnp.int32),
        pltpu.VMEM((_BPW, PADW), jnp.float32),
        pltpu.SemaphoreType.DMA,
    ],
    compiler_params=pltpu.CompilerParams(use_tc_tiling_on_sc=True),
)


_TCOLS = 16384


def _tpose_body(tt, out):
    t = lax.dot_general(tt[...], jnp.eye(EMBED, dtype=jnp.float32),
                        (((0,), (0,)), ((), ())),
                        preferred_element_type=jnp.float32)
    out[...] = jnp.concatenate(
        [t, jnp.zeros((_TCOLS, PADW - EMBED), jnp.float32)], axis=1)


def _make_tpose(nrows):
    return pl.pallas_call(
        _tpose_body,
        grid=(pl.cdiv(nrows, _TCOLS),),
        in_specs=[pl.BlockSpec((EMBED, _TCOLS), lambda i: (0, i))],
        out_specs=pl.BlockSpec((_TCOLS, PADW), lambda i: (i, 0)),
        out_shape=jax.ShapeDtypeStruct((nrows, PADW), jnp.float32),
        compiler_params=pltpu.CompilerParams(
            dimension_semantics=("arbitrary",)),
    )


def _mlp_body(up, ip, w1a, w1b, b1, w2, b2, w3, b3, out):
    h = jnp.dot(up[:, :EMBED], w1a[...], preferred_element_type=jnp.float32)
    h = h + jnp.dot(ip[:, :EMBED], w1b[...], preferred_element_type=jnp.float32)
    h = jnp.maximum(h + b1[...], 0.0)
    h = jnp.maximum(
        jnp.dot(h, w2[...], preferred_element_type=jnp.float32) + b2[...], 0.0)
    o = jnp.dot(h, w3[...], preferred_element_type=jnp.float32)
    out[...] = o[:, 0] + b3[...][0, 0]


_BS = 2048

_mlp = pl.pallas_call(
    _mlp_body,
    grid=(BATCH // _BS,),
    in_specs=[
        pl.BlockSpec((_BS, PADW), lambda i: (i, 0)),
        pl.BlockSpec((_BS, PADW), lambda i: (i, 0)),
        pl.BlockSpec((EMBED, HID1), lambda i: (0, 0)),
        pl.BlockSpec((EMBED, HID1), lambda i: (0, 0)),
        pl.BlockSpec((1, HID1), lambda i: (0, 0)),
        pl.BlockSpec((HID1, HID2), lambda i: (0, 0)),
        pl.BlockSpec((1, HID2), lambda i: (0, 0)),
        pl.BlockSpec((HID2, 1), lambda i: (0, 0)),
        pl.BlockSpec((1, 1), lambda i: (0, 0)),
    ],
    out_specs=pl.BlockSpec((_BS,), lambda i: (i,)),
    out_shape=jax.ShapeDtypeStruct((BATCH,), jnp.float32),
    compiler_params=pltpu.CompilerParams(dimension_semantics=("arbitrary",)),
)


def kernel(users, items, user_table, item_table, W1, b1, W2, b2, W3, b3):
    uidx = users.astype(jnp.int32).reshape(BATCH // CHUNK, CHUNK)
    iidx = items.astype(jnp.int32).reshape(BATCH // CHUNK, CHUNK)
    tpose = _make_tpose(user_table.shape[0])
    tu = tpose(user_table.T)
    ti = tpose(item_table.T)
    up, ip = _gather(uidx, iidx, tu, ti)
    return _mlp(up, ip, W1[:EMBED], W1[EMBED:], b1.reshape(1, HID1),
                W2, b2.reshape(1, HID2), W3, b3.reshape(1, 1))
